# Initial kernel scaffold; baseline (speedup 1.0000x reference)
import jax
import jax.numpy as jnp
from jax.experimental import pallas as pl


def kernel(input_tensor, output_tensor, bias_input, bias_output):
    raise NotImplementedError("write your pallas kernel here")


# speedup vs baseline: 77.2734x; 77.2734x over previous
import jax
import jax.numpy as jnp
from jax.experimental import pallas as pl

_C, _H, _F, _K = 4096, 3, 3, 16
_NBINS = _K ** (_H + _F)
_LANES = 128
_ROWS = _NBINS // _LANES
_BLOCK_ROWS = 4096
_GRID = _ROWS // _BLOCK_ROWS


def _hist_body(xi_ref, xo_ref, bi_ref, bo_ref, out_ref):
    j = pl.program_id(0)
    out_ref[...] = jnp.zeros_like(out_ref)

    @pl.when(j == 0)
    def _():
        def to_idx(x):
            f = x.astype(jnp.int32).astype(jnp.float32)
            f = jnp.minimum(jnp.maximum(f, 0.0), 0.2)
            return f.astype(jnp.int32)

        ii = to_idx(xi_ref[...] + bi_ref[...])
        io = to_idx(xo_ref[...] + bo_ref[...])

        lin = jnp.zeros((_C, _K), jnp.int32)
        for d in range(_H):
            lin = lin + ii[:, d * _K:(d + 1) * _K] * (_K ** (_H + _F - 1 - d))
        for d in range(_F):
            lin = lin + io[:, d * _K:(d + 1) * _K] * (_K ** (_F - 1 - d))

        cols = jax.lax.broadcasted_iota(jnp.int32, (1, _LANES), 1)
        hit = jnp.zeros((_C, _LANES), jnp.float32)
        for k in range(_K):
            hit = jnp.maximum(hit, (lin[:, k:k + 1] == cols).astype(jnp.float32))
        counts = jnp.sum(hit, axis=0, keepdims=True)
        out_ref[0:1, :] = counts * (1.0 / _C)


def kernel(input_tensor, output_tensor, bias_input, bias_output):
    xi = input_tensor.reshape(_C, _H * _K)
    xo = output_tensor.reshape(_C, _F * _K)
    bi = bias_input.reshape(_C, _H * _K)
    bo = bias_output.reshape(_C, _F * _K)

    in_spec = pl.BlockSpec((_C, _H * _K), lambda j: (0, 0))
    flat = pl.pallas_call(
        _hist_body,
        grid=(_GRID,),
        in_specs=[in_spec, in_spec, in_spec, in_spec],
        out_specs=pl.BlockSpec((_BLOCK_ROWS, _LANES), lambda j: (j, 0)),
        out_shape=jax.ShapeDtypeStruct((_ROWS, _LANES), jnp.float32),
    )(xi, xo, bi, bo)
    return flat.reshape((_K,) * (_H + _F))


# speedup vs baseline: 160.3268x; 2.0748x over previous
import jax
import jax.numpy as jnp
from jax.experimental import pallas as pl

_C, _H, _F, _K = 4096, 3, 3, 16


def _hist_body(xi_ref, xo_ref, bi_ref, bo_ref, out_ref):
    i = pl.program_id(0)
    j = pl.program_id(1)
    out_ref[...] = jnp.zeros_like(out_ref)

    @pl.when((i == 0) & (j == 0))
    def _():
        def to_idx(x):
            f = x.astype(jnp.int32).astype(jnp.float32)
            f = jnp.minimum(jnp.maximum(f, 0.0), 0.2)
            return f.astype(jnp.int32)

        ii = to_idx(xi_ref[...] + bi_ref[...])
        io = to_idx(xo_ref[...] + bo_ref[...])

        lin = jnp.zeros((_C, _K), jnp.int32)
        for d in range(_H):
            lin = lin + ii[:, d * _K:(d + 1) * _K] * (_K ** (_H + _F - 1 - d))
        for d in range(_F):
            lin = lin + io[:, d * _K:(d + 1) * _K] * (_K ** (_F - 1 - d))

        hit0 = jnp.max((lin == 0).astype(jnp.float32), axis=1, keepdims=True)
        count = jnp.sum(hit0)
        r = jax.lax.broadcasted_iota(jnp.int32, (8, _K), 0)
        c = jax.lax.broadcasted_iota(jnp.int32, (8, _K), 1)
        tile = jnp.where((r == 0) & (c == 0), count * (1.0 / _C), 0.0)
        out_ref[0, 0, 0, 0, 0:8, 0:_K] = tile


def kernel(input_tensor, output_tensor, bias_input, bias_output):
    xi = input_tensor.reshape(_C, _H * _K)
    xo = output_tensor.reshape(_C, _F * _K)
    bi = bias_input.reshape(_C, _H * _K)
    bo = bias_output.reshape(_C, _F * _K)

    in_spec = pl.BlockSpec((_C, _H * _K), lambda i, j: (0, 0))
    out = pl.pallas_call(
        _hist_body,
        grid=(_K, 4),
        in_specs=[in_spec, in_spec, in_spec, in_spec],
        out_specs=pl.BlockSpec(
            (1, 4, _K, _K, _K, _K), lambda i, j: (i, j, 0, 0, 0, 0)
        ),
        out_shape=jax.ShapeDtypeStruct((_K,) * (_H + _F), jnp.float32),
    )(xi, xo, bi, bo)
    return out
